# baseline (speedup 1.0000x reference)
import functools

import jax
import jax.numpy as jnp
from jax import lax
from jax.experimental import pallas as pl
from jax.experimental.pallas import tpu as pltpu
from jax.experimental.pallas import tpu_sc as plsc

B = 4096
I = 1024
L = 16
NC = 1
NS = 16
NW = NC * NS
ROWS_PER_W = B // NW
CHUNK = 32
NCHUNK = ROWS_PER_W // CHUNK
IDX_SEG = 128
NSEG = I // IDX_SEG


def _irt_body(student_ids_hbm, item_ids_hbm, ability_hbm, difficulty_hbm,
              out_hbm,
              sid_v, sa_v, iid_v, idiff_v, buf0, buf1, gsem, osem0, osem1):
    wid = lax.axis_index("s") * NC + lax.axis_index("c")
    base = wid * ROWS_PER_W

    pltpu.sync_copy(student_ids_hbm.at[pl.ds(base, ROWS_PER_W)], sid_v)
    pltpu.sync_copy(item_ids_hbm, iid_v)

    gathers = [
        pltpu.async_copy(ability_hbm.at[sid_v.at[pl.ds(s * IDX_SEG, IDX_SEG)]],
                         sa_v.at[pl.ds(s * IDX_SEG, IDX_SEG)], gsem)
        for s in range(ROWS_PER_W // IDX_SEG)
    ]
    for j in range(NSEG):
        gathers.append(
            pltpu.async_copy(difficulty_hbm.at[iid_v.at[j]],
                             idiff_v.at[pl.ds(j * IDX_SEG, IDX_SEG)], gsem))
    for cp in gathers:
        cp.wait()

    def splat(vec, lane):
        return lax.gather(
            vec, jnp.full((L, 1), lane, jnp.int32),
            dimension_numbers=lax.GatherDimensionNumbers(
                offset_dims=(), collapsed_slice_dims=(0,),
                start_index_map=(0,)),
            slice_sizes=(1,),
            mode=lax.GatherScatterMode.PROMISE_IN_BOUNDS)

    UN = 4
    bufs = (buf0, buf1)
    osems = (osem0, osem1)
    pending = [None, None]
    for g in range(NCHUNK):
        sel = g % 2
        buf = bufs[sel]
        if pending[sel] is not None:
            pending[sel].wait()

        for grp in range(CHUNK // L):
            sa16 = sa_v[pl.ds(g * CHUNK + grp * L, L)]
            splats = tuple(splat(sa16, j) for j in range(L))

            def col_body(c, sabs, grp=grp, buf=buf):
                base = c * (UN * L)
                diffs = [idiff_v[pl.ds(base + u * L, L)] for u in range(UN)]
                for u in range(UN):
                    for j in range(L):
                        buf[grp * L + j, pl.ds(base + u * L, L)] = (
                            sabs[j] - diffs[u])
                return sabs

            lax.fori_loop(0, I // (UN * L), col_body, splats)
        pending[sel] = pltpu.async_copy(
            buf, out_hbm.at[pl.ds(base + g * CHUNK, CHUNK)], osems[sel])
    for cp in pending:
        if cp is not None:
            cp.wait()


@functools.partial(jax.jit, static_argnums=())
def _irt_sc(student_ids, item_ids2, student_ability, item_difficulty):
    mesh = plsc.VectorSubcoreMesh(core_axis_name="c", subcore_axis_name="s",
                                  num_cores=NC)
    run = pl.kernel(
        _irt_body,
        mesh=mesh,
        out_type=jax.ShapeDtypeStruct((B, I), jnp.float32),
        scratch_types=[
            pltpu.VMEM((ROWS_PER_W,), jnp.int32),
            pltpu.VMEM((ROWS_PER_W,), jnp.float32),
            pltpu.VMEM((NSEG, IDX_SEG), jnp.int32),
            pltpu.VMEM((I,), jnp.float32),
            pltpu.VMEM((CHUNK, I), jnp.float32),
            pltpu.VMEM((CHUNK, I), jnp.float32),
            pltpu.SemaphoreType.DMA,
            pltpu.SemaphoreType.DMA,
            pltpu.SemaphoreType.DMA,
        ],
    )
    return run(student_ids, item_ids2, student_ability, item_difficulty)


def kernel(student_ids, item_ids, student_ability, item_difficulty):
    item_ids2 = item_ids.reshape(NSEG, IDX_SEG)
    return _irt_sc(student_ids.astype(jnp.int32), item_ids2.astype(jnp.int32),
                   student_ability, item_difficulty)


# speedup vs baseline: 1.4105x; 1.4105x over previous
import functools

import jax
import jax.numpy as jnp
from jax import lax
from jax.experimental import pallas as pl
from jax.experimental.pallas import tpu as pltpu
from jax.experimental.pallas import tpu_sc as plsc

B = 4096
I = 1024
NC = 1
NS = 16
NW = NC * NS
SEG = 128
ROWS_PER_W = B // NW

TC_BLOCK = 4096


def _gather_body(student_ids_hbm, item_ids_hbm, ability_hbm, difficulty_hbm,
                 sa_out_hbm, idiff_out_hbm,
                 sid_v, sa_v, iid_v, idiff_v, sem):
    wid = lax.axis_index("s") * NC + lax.axis_index("c")
    base = wid * ROWS_PER_W

    pltpu.sync_copy(student_ids_hbm.at[pl.ds(base, ROWS_PER_W)], sid_v)
    cps = [
        pltpu.async_copy(ability_hbm.at[sid_v.at[pl.ds(s * SEG, SEG)]],
                         sa_v.at[pl.ds(s * SEG, SEG)], sem)
        for s in range(ROWS_PER_W // SEG)
    ]
    for cp in cps:
        cp.wait()
    pltpu.sync_copy(sa_v, sa_out_hbm.at[pl.ds(base, ROWS_PER_W)])

    @pl.when(wid < I // SEG)
    def _():
        ibase = wid * SEG
        pltpu.sync_copy(item_ids_hbm.at[pl.ds(ibase, SEG)], iid_v)
        cp2 = pltpu.async_copy(difficulty_hbm.at[iid_v], idiff_v, sem)
        cp2.wait()
        pltpu.sync_copy(idiff_v, idiff_out_hbm.at[pl.ds(ibase, SEG)])


def _sc_gather(student_ids, item_ids, student_ability, item_difficulty):
    mesh = plsc.VectorSubcoreMesh(core_axis_name="c", subcore_axis_name="s",
                                  num_cores=NC)
    run = pl.kernel(
        _gather_body,
        mesh=mesh,
        out_type=(jax.ShapeDtypeStruct((B,), jnp.float32),
                  jax.ShapeDtypeStruct((I,), jnp.float32)),
        scratch_types=[
            pltpu.VMEM((ROWS_PER_W,), jnp.int32),
            pltpu.VMEM((ROWS_PER_W,), jnp.float32),
            pltpu.VMEM((SEG,), jnp.int32),
            pltpu.VMEM((SEG,), jnp.float32),
            pltpu.SemaphoreType.DMA,
        ],
    )
    return run(student_ids, item_ids, student_ability, item_difficulty)


def _dense_body(sa_ref, idiff_ref, out_ref):
    i = pl.program_id(0)
    sa_c = sa_ref[pl.ds(i * TC_BLOCK, TC_BLOCK)]
    out_ref[...] = sa_c[:, None] - idiff_ref[...][None, :]


def _tc_dense(sa, idiff):
    return pl.pallas_call(
        _dense_body,
        grid=(B // TC_BLOCK,),
        in_specs=[
            pl.BlockSpec((B,), lambda i: (0,)),
            pl.BlockSpec((I,), lambda i: (0,)),
        ],
        out_specs=pl.BlockSpec((TC_BLOCK, I), lambda i: (i, 0)),
        out_shape=jax.ShapeDtypeStruct((B, I), jnp.float32),
    )(sa, idiff)


@jax.jit
def _irt(student_ids, item_ids, student_ability, item_difficulty):
    sa, idiff = _sc_gather(student_ids, item_ids,
                           student_ability, item_difficulty)
    return _tc_dense(sa, idiff)


def kernel(student_ids, item_ids, student_ability, item_difficulty):
    return _irt(student_ids.astype(jnp.int32), item_ids.astype(jnp.int32),
                student_ability, item_difficulty)


# speedup vs baseline: 1.4971x; 1.0613x over previous
import functools

import jax
import jax.numpy as jnp
from jax import lax
from jax.experimental import pallas as pl
from jax.experimental.pallas import tpu as pltpu
from jax.experimental.pallas import tpu_sc as plsc

B = 4096
I = 1024
NC = 1
NS = 16
NW = NC * NS
SEG = 128
ROWS_PER_W = B // NW

TC_BLOCK = 1024


def _gather_body(student_ids_hbm, item_ids_hbm, ability_hbm, difficulty_hbm,
                 sa_out_hbm, idiff_out_hbm,
                 sid_v, sa_v, iid_v, idiff_v, sem, isem):
    wid = lax.axis_index("s") * NC + lax.axis_index("c")
    base = wid * ROWS_PER_W

    nseg = ROWS_PER_W // SEG
    do_items = wid < I // SEG
    ibase = wid * SEG

    pltpu.sync_copy(student_ids_hbm.at[pl.ds(base, ROWS_PER_W)], sid_v)

    @pl.when(do_items)
    def _():
        pltpu.sync_copy(item_ids_hbm.at[pl.ds(ibase, SEG)], iid_v)

    cps = [
        pltpu.async_copy(ability_hbm.at[sid_v.at[pl.ds(s * SEG, SEG)]],
                         sa_v.at[pl.ds(s * SEG, SEG)], sem)
        for s in range(nseg)
    ]

    @pl.when(do_items)
    def _():
        pltpu.async_copy(difficulty_hbm.at[iid_v], idiff_v, isem).wait()
        pltpu.sync_copy(idiff_v, idiff_out_hbm.at[pl.ds(ibase, SEG)])

    for cp in cps:
        cp.wait()
    pltpu.sync_copy(sa_v, sa_out_hbm.at[pl.ds(base, ROWS_PER_W)])


def _sc_gather(student_ids, item_ids, student_ability, item_difficulty):
    mesh = plsc.VectorSubcoreMesh(core_axis_name="c", subcore_axis_name="s",
                                  num_cores=NC)
    run = pl.kernel(
        _gather_body,
        mesh=mesh,
        out_type=(jax.ShapeDtypeStruct((B,), jnp.float32),
                  jax.ShapeDtypeStruct((I,), jnp.float32)),
        scratch_types=[
            pltpu.VMEM((ROWS_PER_W,), jnp.int32),
            pltpu.VMEM((ROWS_PER_W,), jnp.float32),
            pltpu.VMEM((SEG,), jnp.int32),
            pltpu.VMEM((SEG,), jnp.float32),
            pltpu.SemaphoreType.DMA,
            pltpu.SemaphoreType.DMA,
        ],
    )
    return run(student_ids, item_ids, student_ability, item_difficulty)


def _dense_body(sa_ref, idiff_ref, out_ref):
    i = pl.program_id(0)
    sa_c = sa_ref[pl.ds(i * TC_BLOCK, TC_BLOCK)]
    out_ref[...] = sa_c[:, None] - idiff_ref[...][None, :]


def _tc_dense(sa, idiff):
    return pl.pallas_call(
        _dense_body,
        grid=(B // TC_BLOCK,),
        in_specs=[
            pl.BlockSpec((B,), lambda i: (0,)),
            pl.BlockSpec((I,), lambda i: (0,)),
        ],
        out_specs=pl.BlockSpec((TC_BLOCK, I), lambda i: (i, 0)),
        out_shape=jax.ShapeDtypeStruct((B, I), jnp.float32),
    )(sa, idiff)


@jax.jit
def _irt(student_ids, item_ids, student_ability, item_difficulty):
    sa, idiff = _sc_gather(student_ids, item_ids,
                           student_ability, item_difficulty)
    return _tc_dense(sa, idiff)


def kernel(student_ids, item_ids, student_ability, item_difficulty):
    return _irt(student_ids.astype(jnp.int32), item_ids.astype(jnp.int32),
                student_ability, item_difficulty)


# speedup vs baseline: 1.5016x; 1.0030x over previous
import functools

import jax
import jax.numpy as jnp
from jax import lax
from jax.experimental import pallas as pl
from jax.experimental.pallas import tpu as pltpu
from jax.experimental.pallas import tpu_sc as plsc

B = 4096
I = 1024
NC = 1
NS = 16
NW = NC * NS
SEG = 128
ROWS_PER_W = B // NW

TC_BLOCK = 1024


def _gather_body(student_ids_hbm, item_ids_hbm, ability_hbm, difficulty_hbm,
                 sa_out_hbm, idiff_out_hbm,
                 sid_v, sa_v, iid_v, idiff_v, sem, isem):
    wid = lax.axis_index("s") * NC + lax.axis_index("c")
    base = wid * ROWS_PER_W

    nseg = ROWS_PER_W // SEG
    do_items = wid < I // SEG
    ibase = wid * SEG

    pltpu.sync_copy(student_ids_hbm.at[pl.ds(base, ROWS_PER_W)], sid_v)

    @pl.when(do_items)
    def _():
        pltpu.sync_copy(item_ids_hbm.at[pl.ds(ibase, SEG)], iid_v)

    cps = [
        pltpu.async_copy(ability_hbm.at[sid_v.at[pl.ds(s * SEG, SEG)]],
                         sa_v.at[pl.ds(s * SEG, SEG)], sem)
        for s in range(nseg)
    ]

    @pl.when(do_items)
    def _():
        pltpu.async_copy(difficulty_hbm.at[iid_v], idiff_v, isem).wait()
        pltpu.sync_copy(idiff_v, idiff_out_hbm.at[pl.ds(ibase, SEG)])

    wcps = []
    for s in range(nseg):
        cps[s].wait()
        wcps.append(
            pltpu.async_copy(sa_v.at[pl.ds(s * SEG, SEG)],
                             sa_out_hbm.at[pl.ds(base + s * SEG, SEG)], isem))
    for cp in wcps:
        cp.wait()


def _sc_gather(student_ids, item_ids, student_ability, item_difficulty):
    mesh = plsc.VectorSubcoreMesh(core_axis_name="c", subcore_axis_name="s",
                                  num_cores=NC)
    run = pl.kernel(
        _gather_body,
        mesh=mesh,
        out_type=(jax.ShapeDtypeStruct((B,), jnp.float32),
                  jax.ShapeDtypeStruct((I,), jnp.float32)),
        scratch_types=[
            pltpu.VMEM((ROWS_PER_W,), jnp.int32),
            pltpu.VMEM((ROWS_PER_W,), jnp.float32),
            pltpu.VMEM((SEG,), jnp.int32),
            pltpu.VMEM((SEG,), jnp.float32),
            pltpu.SemaphoreType.DMA,
            pltpu.SemaphoreType.DMA,
        ],
    )
    return run(student_ids, item_ids, student_ability, item_difficulty)


def _dense_body(sa_ref, idiff_ref, out_ref):
    i = pl.program_id(0)
    sa_c = sa_ref[pl.ds(i * TC_BLOCK, TC_BLOCK)]
    out_ref[...] = sa_c[:, None] - idiff_ref[...][None, :]


def _tc_dense(sa, idiff):
    return pl.pallas_call(
        _dense_body,
        grid=(B // TC_BLOCK,),
        in_specs=[
            pl.BlockSpec((B,), lambda i: (0,)),
            pl.BlockSpec((I,), lambda i: (0,)),
        ],
        out_specs=pl.BlockSpec((TC_BLOCK, I), lambda i: (i, 0)),
        out_shape=jax.ShapeDtypeStruct((B, I), jnp.float32),
    )(sa, idiff)


@jax.jit
def _irt(student_ids, item_ids, student_ability, item_difficulty):
    sa, idiff = _sc_gather(student_ids, item_ids,
                           student_ability, item_difficulty)
    return _tc_dense(sa, idiff)


def kernel(student_ids, item_ids, student_ability, item_difficulty):
    return _irt(student_ids.astype(jnp.int32), item_ids.astype(jnp.int32),
                student_ability, item_difficulty)
